# baseline (speedup 1.0000x reference)
import dataclasses
import functools

import jax
import jax.numpy as jnp
from jax import lax
from jax.experimental import pallas as pl
from jax.experimental.pallas import tpu as pltpu
from jax.experimental.pallas import tpu_sc as plsc

N = 100000
D = 128
B = 16384
NC = 2
NS = 16
L = 16

NCHUNK = 20
R = N // NCHUNK
KPC = NCHUNK // NC
DENSE_TILES = 5
ROWS_PER_TILE = R // DENSE_TILES
SCAN_PER_TILE = B // NS
NVEC = SCAN_PER_TILE // L
KB = 128
MAXM = SCAN_PER_TILE + KB
NBROWS = MAXM // KB


def _sc_index_add(dst, src, idx):
  mesh = plsc.VectorSubcoreMesh(
      core_axis_name="c", subcore_axis_name="s",
      num_cores=NC, num_subcores=NS)
  cp = pltpu.CompilerParams()
  if "needs_layout_passes" in pltpu.CompilerParams.__dataclass_fields__:
    cp = dataclasses.replace(cp, needs_layout_passes=False)

  @functools.partial(
      pl.kernel,
      out_type=jax.ShapeDtypeStruct((N, D), jnp.float32),
      mesh=mesh,
      compiler_params=cp,
      scratch_types=[
          pltpu.VMEM_SHARED((R + L, D), jnp.float32),
          pltpu.VMEM_SHARED((R + L, D), jnp.float32),
          pltpu.VMEM((SCAN_PER_TILE,), jnp.int32),
          pltpu.VMEM((MAXM,), jnp.int32),
          pltpu.VMEM((NBROWS, KB), jnp.int32),
          pltpu.VMEM((KB, D), jnp.float32),
          pltpu.SemaphoreType.DMA,
          pltpu.SemaphoreType.DMA,
          pltpu.SemaphoreType.DMA,
          pltpu.SemaphoreType.DMA,
      ],
  )
  def run(dst_hbm, src_hbm, idx_hbm, out_hbm,
          acc0, acc1, idxbuf, posbuf, lidx2d, staging,
          lsem0, lsem1, ssem0, ssem1):
    core = lax.axis_index("c")
    sub = lax.axis_index("s")
    lanes = lax.iota(jnp.int32, L)

    def hbm_slc(k_local):
      base = (k_local * NC + core) * R
      return dst_hbm.at[pl.ds(base + sub * ROWS_PER_TILE, ROWS_PER_TILE)]

    def out_slc(k_local):
      base = (k_local * NC + core) * R
      return out_hbm.at[pl.ds(base + sub * ROWS_PER_TILE, ROWS_PER_TILE)]

    def acc_slc(acc):
      return acc.at[pl.ds(sub * ROWS_PER_TILE, ROWS_PER_TILE)]

    def load_issue(k_local, acc, sem):
      pltpu.async_copy(hbm_slc(k_local), acc_slc(acc), sem)

    def load_wait(k_local, acc, sem):
      pltpu.make_async_copy(hbm_slc(k_local), acc_slc(acc), sem).wait()

    def store_issue(k_local, acc, sem):
      pltpu.async_copy(acc_slc(acc), out_slc(k_local), sem)

    def store_wait(k_local, acc, sem):
      pltpu.make_async_copy(acc_slc(acc), out_slc(k_local), sem).wait()

    def work(k_local, acc):
      base = (k_local * NC + core) * R
      ones = lanes >= 0

      def scan_body(v, m_vec):
        vec = idxbuf[pl.ds(v * L, L)]
        rel = vec - base
        mask = rel.astype(jnp.uint32) < jnp.uint32(R)
        mi = mask.astype(jnp.int32)
        off = m_vec + plsc.cumsum(mi) - mi
        pos = lanes + (sub * SCAN_PER_TILE + v * L)
        plsc.store_scatter(posbuf, [off], pos, mask=mask)
        plsc.store_scatter(lidx2d, [off >> 7, off & (KB - 1)], rel, mask=mask)
        return m_vec + plsc.all_reduce_population_count(mask)

      with jax.named_scope("ph_scan"):
        m_vec = lax.fori_loop(0, NVEC, scan_body, jnp.zeros((L,), jnp.int32),
                              unroll=4)
        m = jnp.max(m_vec)

      @pl.loop(0, KB // L)
      def _pad(j):
        off_pad = m + lanes + j * L
        plsc.store_scatter(posbuf, [off_pad], lanes + j * L, mask=ones)
        plsc.store_scatter(lidx2d, [off_pad >> 7, off_pad & (KB - 1)],
                           lanes + R, mask=ones)

      nb = (m + (KB - 1)) // KB

      def batch_body(b, carry):
        pltpu.sync_copy(src_hbm.at[posbuf.at[pl.ds(b * KB, KB)]], staging)
        pltpu.sync_copy(staging, acc.at[lidx2d.at[b]], add=True)
        return carry

      with jax.named_scope("ph_batches"):
        lax.fori_loop(0, nb, batch_body, jnp.int32(0))

    pltpu.sync_copy(idx_hbm.at[pl.ds(sub * SCAN_PER_TILE, SCAN_PER_TILE)],
                    idxbuf)

    @pl.when(sub < DENSE_TILES)
    def _prime():
      load_issue(0, acc0, lsem0)
      load_issue(1, acc1, lsem1)

    @pl.loop(0, KPC // 2)
    def _pair(j):
      k0 = 2 * j
      k1 = 2 * j + 1

      with jax.named_scope("ph_ldwait0"):
        @pl.when(sub < DENSE_TILES)
        def _w0():
          load_wait(k0, acc0, lsem0)
        plsc.subcore_barrier()
      with jax.named_scope("ph_work0"):
        work(k0, acc0)
      with jax.named_scope("ph_bar0"):
        plsc.subcore_barrier()

      @pl.when(sub < DENSE_TILES)
      def _s0():
        store_issue(k0, acc0, ssem0)

      with jax.named_scope("ph_ldwait1"):
        @pl.when(sub < DENSE_TILES)
        def _w1():
          load_wait(k1, acc1, lsem1)
        plsc.subcore_barrier()
      with jax.named_scope("ph_work1"):
        work(k1, acc1)
      with jax.named_scope("ph_bar1"):
        plsc.subcore_barrier()

      @pl.when(sub < DENSE_TILES)
      def _s1():
        store_issue(k1, acc1, ssem1)

      with jax.named_scope("ph_reissue"):
        @pl.when(jnp.logical_and(sub < DENSE_TILES, j < KPC // 2 - 1))
        def _reissue():
          store_wait(k0, acc0, ssem0)
          load_issue(k0 + 2, acc0, lsem0)
          store_wait(k1, acc1, ssem1)
          load_issue(k1 + 2, acc1, lsem1)

    @pl.when(sub < DENSE_TILES)
    def _drain():
      store_wait(KPC - 2, acc0, ssem0)
      store_wait(KPC - 1, acc1, ssem1)

  return run(dst, src, idx)


def kernel(dst_tensor, src_tensor, index_tensor):
  return _sc_index_add(dst_tensor, src_tensor,
                       index_tensor.astype(jnp.int32))


# speedup vs baseline: 1.0808x; 1.0808x over previous
import dataclasses
import functools

import jax
import jax.numpy as jnp
from jax import lax
from jax.experimental import pallas as pl
from jax.experimental.pallas import tpu as pltpu
from jax.experimental.pallas import tpu_sc as plsc

N = 100000
D = 128
B = 16384
NC = 2
NS = 16
L = 16

NCHUNK = 20
R = N // NCHUNK
KPC = NCHUNK // NC
DENSE_TILES = 5
ROWS_PER_TILE = R // DENSE_TILES
SCAN_PER_TILE = B // NS
NVEC = SCAN_PER_TILE // L
KB = 64
KBSH = KB.bit_length() - 1
MAXM = SCAN_PER_TILE + KB
NBROWS = MAXM // KB


def _sc_index_add(dst, src, idx):
  mesh = plsc.VectorSubcoreMesh(
      core_axis_name="c", subcore_axis_name="s",
      num_cores=NC, num_subcores=NS)
  cp = pltpu.CompilerParams()
  if "needs_layout_passes" in pltpu.CompilerParams.__dataclass_fields__:
    cp = dataclasses.replace(cp, needs_layout_passes=False)

  @functools.partial(
      pl.kernel,
      out_type=jax.ShapeDtypeStruct((N, D), jnp.float32),
      mesh=mesh,
      compiler_params=cp,
      scratch_types=[
          pltpu.VMEM_SHARED((R + L, D), jnp.float32),
          pltpu.VMEM_SHARED((R + L, D), jnp.float32),
          pltpu.VMEM((SCAN_PER_TILE,), jnp.int32),
          pltpu.VMEM((MAXM,), jnp.int32),
          pltpu.VMEM((NBROWS, KB), jnp.int32),
          pltpu.VMEM((KB, D), jnp.float32),
          pltpu.SemaphoreType.DMA,
          pltpu.SemaphoreType.DMA,
          pltpu.SemaphoreType.DMA,
          pltpu.SemaphoreType.DMA,
      ],
  )
  def run(dst_hbm, src_hbm, idx_hbm, out_hbm,
          acc0, acc1, idxbuf, posbuf, lidx2d, staging,
          lsem0, lsem1, ssem0, ssem1):
    core = lax.axis_index("c")
    sub = lax.axis_index("s")
    lanes = lax.iota(jnp.int32, L)

    def hbm_slc(k_local):
      base = (k_local * NC + core) * R
      return dst_hbm.at[pl.ds(base + sub * ROWS_PER_TILE, ROWS_PER_TILE)]

    def out_slc(k_local):
      base = (k_local * NC + core) * R
      return out_hbm.at[pl.ds(base + sub * ROWS_PER_TILE, ROWS_PER_TILE)]

    def acc_slc(acc):
      return acc.at[pl.ds(sub * ROWS_PER_TILE, ROWS_PER_TILE)]

    def load_issue(k_local, acc, sem):
      pltpu.async_copy(hbm_slc(k_local), acc_slc(acc), sem)

    def load_wait(k_local, acc, sem):
      pltpu.make_async_copy(hbm_slc(k_local), acc_slc(acc), sem).wait()

    def store_issue(k_local, acc, sem):
      pltpu.async_copy(acc_slc(acc), out_slc(k_local), sem)

    def store_wait(k_local, acc, sem):
      pltpu.make_async_copy(acc_slc(acc), out_slc(k_local), sem).wait()

    def work(k_local, acc):
      base = (k_local * NC + core) * R
      ones = lanes >= 0

      def scan_body(v, m_vec):
        vec = idxbuf[pl.ds(v * L, L)]
        rel = vec - base
        mask = rel.astype(jnp.uint32) < jnp.uint32(R)
        mi = mask.astype(jnp.int32)
        off = m_vec + plsc.cumsum(mi) - mi
        pos = lanes + (sub * SCAN_PER_TILE + v * L)
        plsc.store_scatter(posbuf, [off], pos, mask=mask)
        plsc.store_scatter(lidx2d, [off >> KBSH, off & (KB - 1)], rel,
                           mask=mask)
        return m_vec + plsc.all_reduce_population_count(mask)

      with jax.named_scope("ph_scan"):
        m_vec = lax.fori_loop(0, NVEC, scan_body, jnp.zeros((L,), jnp.int32),
                              unroll=4)
        m = jnp.max(m_vec)

      @pl.loop(0, KB // L)
      def _pad(j):
        off_pad = m + lanes + j * L
        plsc.store_scatter(posbuf, [off_pad], lanes + j * L, mask=ones)
        plsc.store_scatter(lidx2d, [off_pad >> KBSH, off_pad & (KB - 1)],
                           lanes + R, mask=ones)

      nb = (m + (KB - 1)) // KB

      def batch_body(b, carry):
        pltpu.sync_copy(src_hbm.at[posbuf.at[pl.ds(b * KB, KB)]], staging)
        pltpu.sync_copy(staging, acc.at[lidx2d.at[b]], add=True)
        return carry

      with jax.named_scope("ph_batches"):
        lax.fori_loop(0, nb, batch_body, jnp.int32(0))

    pltpu.sync_copy(idx_hbm.at[pl.ds(sub * SCAN_PER_TILE, SCAN_PER_TILE)],
                    idxbuf)

    @pl.when(sub < DENSE_TILES)
    def _prime():
      load_issue(0, acc0, lsem0)
      load_issue(1, acc1, lsem1)

    @pl.loop(0, KPC // 2)
    def _pair(j):
      k0 = 2 * j
      k1 = 2 * j + 1

      with jax.named_scope("ph_ldwait0"):
        @pl.when(sub < DENSE_TILES)
        def _w0():
          load_wait(k0, acc0, lsem0)
        plsc.subcore_barrier()
      with jax.named_scope("ph_work0"):
        work(k0, acc0)
      with jax.named_scope("ph_bar0"):
        plsc.subcore_barrier()

      @pl.when(sub < DENSE_TILES)
      def _s0():
        store_issue(k0, acc0, ssem0)

      with jax.named_scope("ph_ldwait1"):
        @pl.when(sub < DENSE_TILES)
        def _w1():
          load_wait(k1, acc1, lsem1)
        plsc.subcore_barrier()
      with jax.named_scope("ph_work1"):
        work(k1, acc1)
      with jax.named_scope("ph_bar1"):
        plsc.subcore_barrier()

      @pl.when(sub < DENSE_TILES)
      def _s1():
        store_issue(k1, acc1, ssem1)

      with jax.named_scope("ph_reissue"):
        @pl.when(jnp.logical_and(sub < DENSE_TILES, j < KPC // 2 - 1))
        def _reissue():
          store_wait(k0, acc0, ssem0)
          load_issue(k0 + 2, acc0, lsem0)
          store_wait(k1, acc1, ssem1)
          load_issue(k1 + 2, acc1, lsem1)

    @pl.when(sub < DENSE_TILES)
    def _drain():
      store_wait(KPC - 2, acc0, ssem0)
      store_wait(KPC - 1, acc1, ssem1)

  return run(dst, src, idx)


def kernel(dst_tensor, src_tensor, index_tensor):
  return _sc_index_add(dst_tensor, src_tensor,
                       index_tensor.astype(jnp.int32))


# speedup vs baseline: 1.1978x; 1.1083x over previous
import dataclasses
import functools

import jax
import jax.numpy as jnp
from jax import lax
from jax.experimental import pallas as pl
from jax.experimental.pallas import tpu as pltpu
from jax.experimental.pallas import tpu_sc as plsc

N = 100000
D = 128
B = 16384
NC = 2
NS = 16
L = 16

NCHUNK = 20
R = N // NCHUNK
KPC = NCHUNK // NC
DENSE_TILES = 5
ROWS_PER_TILE = R // DENSE_TILES
SCAN_PER_TILE = B // NS
NVEC = SCAN_PER_TILE // L
KB = 64
KBSH = KB.bit_length() - 1
MAXM = SCAN_PER_TILE + KB
NBROWS = MAXM // KB


def _sc_index_add(dst, src, idx):
  mesh = plsc.VectorSubcoreMesh(
      core_axis_name="c", subcore_axis_name="s",
      num_cores=NC, num_subcores=NS)
  cp = pltpu.CompilerParams()
  if "needs_layout_passes" in pltpu.CompilerParams.__dataclass_fields__:
    cp = dataclasses.replace(cp, needs_layout_passes=False)

  @functools.partial(
      pl.kernel,
      out_type=jax.ShapeDtypeStruct((N, D), jnp.float32),
      mesh=mesh,
      compiler_params=cp,
      scratch_types=[
          pltpu.VMEM_SHARED((R + L, D), jnp.float32),
          pltpu.VMEM_SHARED((R + L, D), jnp.float32),
          pltpu.VMEM((SCAN_PER_TILE,), jnp.int32),
          pltpu.VMEM((MAXM,), jnp.int32),
          pltpu.VMEM((MAXM,), jnp.int32),
          pltpu.VMEM((NBROWS, KB), jnp.int32),
          pltpu.VMEM((NBROWS, KB), jnp.int32),
          pltpu.VMEM((KB, D), jnp.float32),
          pltpu.VMEM((KB, D), jnp.float32),
          pltpu.SMEM((2,), jnp.int32),
          pltpu.SemaphoreType.DMA,
          pltpu.SemaphoreType.DMA,
          pltpu.SemaphoreType.DMA,
          pltpu.SemaphoreType.DMA,
          pltpu.SemaphoreType.DMA,
          pltpu.SemaphoreType.DMA,
      ],
  )
  def run(dst_hbm, src_hbm, idx_hbm, out_hbm,
          acc0, acc1, idxbuf, posA, posB, lidxA, lidxB, stgA, stgB, msc,
          lsem0, lsem1, ssem0, ssem1, gsemA, gsemB):
    core = lax.axis_index("c")
    sub = lax.axis_index("s")
    lanes = lax.iota(jnp.int32, L)
    ones = lanes >= 0

    def hbm_slc(k_local):
      base = (k_local * NC + core) * R
      return dst_hbm.at[pl.ds(base + sub * ROWS_PER_TILE, ROWS_PER_TILE)]

    def out_slc(k_local):
      base = (k_local * NC + core) * R
      return out_hbm.at[pl.ds(base + sub * ROWS_PER_TILE, ROWS_PER_TILE)]

    def acc_slc(acc):
      return acc.at[pl.ds(sub * ROWS_PER_TILE, ROWS_PER_TILE)]

    def load_issue(k_local, acc, sem):
      pltpu.async_copy(hbm_slc(k_local), acc_slc(acc), sem)

    def load_wait(k_local, acc, sem):
      pltpu.make_async_copy(hbm_slc(k_local), acc_slc(acc), sem).wait()

    def store_issue(k_local, acc, sem):
      pltpu.async_copy(acc_slc(acc), out_slc(k_local), sem)

    def store_wait(k_local, acc, sem):
      pltpu.make_async_copy(acc_slc(acc), out_slc(k_local), sem).wait()

    def scan_chunk(k_local, pos, lidx, stg, gsem, par):
      base = (k_local * NC + core) * R

      def scan_body(v, m_vec):
        vec = idxbuf[pl.ds(v * L, L)]
        rel = vec - base
        mask = rel.astype(jnp.uint32) < jnp.uint32(R)
        mi = mask.astype(jnp.int32)
        off = m_vec + plsc.cumsum(mi) - mi
        p = lanes + (sub * SCAN_PER_TILE + v * L)
        plsc.store_scatter(pos, [off], p, mask=mask)
        plsc.store_scatter(lidx, [off >> KBSH, off & (KB - 1)], rel,
                           mask=mask)
        return m_vec + plsc.all_reduce_population_count(mask)

      m_vec = lax.fori_loop(0, NVEC, scan_body, jnp.zeros((L,), jnp.int32),
                            unroll=4)
      m = jnp.max(m_vec)
      msc[par] = m

      @pl.loop(0, KB // L)
      def _pad(j):
        off_pad = m + lanes + j * L
        plsc.store_scatter(pos, [off_pad], lanes + j * L, mask=ones)
        plsc.store_scatter(lidx, [off_pad >> KBSH, off_pad & (KB - 1)],
                           lanes + R, mask=ones)

      pltpu.async_copy(src_hbm.at[pos.at[pl.ds(0, KB)]], stg, gsem)

    def add_phase(acc, pos, lidx, stg, gsem, par):
      pltpu.make_async_copy(src_hbm.at[pos.at[pl.ds(0, KB)]], stg,
                            gsem).wait()
      pltpu.sync_copy(stg, acc.at[lidx.at[0]], add=True)
      nb = (msc[par] + (KB - 1)) >> KBSH

      def batch_body(b, carry):
        pltpu.sync_copy(src_hbm.at[pos.at[pl.ds(b * KB, KB)]], stg)
        pltpu.sync_copy(stg, acc.at[lidx.at[b]], add=True)
        return carry

      lax.fori_loop(1, nb, batch_body, jnp.int32(0))

    sets = ((posA, lidxA, stgA, gsemA), (posB, lidxB, stgB, gsemB))

    def slot(k, acc, lsem, ssem, par):
      @pl.when(jnp.int32(k + 1) < KPC)
      def _prefetch():
        scan_chunk(k + 1, *sets[1 - par], 1 - par)

      @pl.when(sub < DENSE_TILES)
      def _w():
        load_wait(k, acc, lsem)
      plsc.subcore_barrier()
      add_phase(acc, *sets[par], par)
      plsc.subcore_barrier()

      @pl.when(sub < DENSE_TILES)
      def _s():
        store_issue(k, acc, ssem)

    @pl.when(sub < DENSE_TILES)
    def _prime():
      load_issue(0, acc0, lsem0)
      load_issue(1, acc1, lsem1)

    pltpu.sync_copy(idx_hbm.at[pl.ds(sub * SCAN_PER_TILE, SCAN_PER_TILE)],
                    idxbuf)
    scan_chunk(0, *sets[0], 0)

    @pl.loop(0, KPC // 2)
    def _pair(j):
      k0 = 2 * j
      k1 = 2 * j + 1
      slot(k0, acc0, lsem0, ssem0, 0)
      slot(k1, acc1, lsem1, ssem1, 1)

      @pl.when(jnp.logical_and(sub < DENSE_TILES, j < KPC // 2 - 1))
      def _reissue():
        store_wait(k0, acc0, ssem0)
        load_issue(k0 + 2, acc0, lsem0)
        store_wait(k1, acc1, ssem1)
        load_issue(k1 + 2, acc1, lsem1)

    @pl.when(sub < DENSE_TILES)
    def _drain():
      store_wait(KPC - 2, acc0, ssem0)
      store_wait(KPC - 1, acc1, ssem1)

  return run(dst, src, idx)


def kernel(dst_tensor, src_tensor, index_tensor):
  return _sc_index_add(dst_tensor, src_tensor,
                       index_tensor.astype(jnp.int32))
